# baseline (speedup 1.0000x reference)
import functools

import jax
import jax.numpy as jnp
from jax import lax
from jax.experimental import pallas as pl
from jax.experimental.pallas import tpu as pltpu
from jax.experimental.pallas import tpu_sc as plsc

_N = 65536
_D = 32
_K = 512
_BLK = 2048

_NC = 2
_NS = 16
_NW = _NC * _NS
_BPW = _N // _NW
_CHUNK = 128
_NCHUNK = _BPW // _CHUNK


def _tc_body(x_ref, v_ref, idx_ref, idx2_ref, vt_ref, loss_ref):
    xb = x_ref[...]
    v = v_ref[...]
    xv = jnp.dot(xb, v, preferred_element_type=jnp.float32)
    rownorm = jnp.sum(xb * xb, axis=1, keepdims=True)
    vnorm = jnp.sum(v * v, axis=0, keepdims=True)
    d = (rownorm - 2.0 * xv) + vnorm
    m = jnp.min(d, axis=1, keepdims=True)
    iota = lax.broadcasted_iota(jnp.int32, d.shape, 1)
    idx = jnp.min(jnp.where(d == m, iota, _K), axis=1)
    idx_ref[...] = idx[:, None]
    idx2_ref[...] = idx.reshape(_BLK // _CHUNK, _CHUNK)

    @pl.when(pl.program_id(0) == 0)
    def _():
        loss_ref[0] = 0.0
        vt_ref[...] = jnp.concatenate(
            [v.T, jnp.zeros((_K, 128 - _D), jnp.float32)], axis=1)

    loss_ref[0] += jnp.sum(m)


def _tc_part(x, vectors):
    grid = _N // _BLK
    return pl.pallas_call(
        _tc_body,
        grid=(grid,),
        in_specs=[
            pl.BlockSpec((_BLK, _D), lambda i: (i, 0)),
            pl.BlockSpec((_D, _K), lambda i: (0, 0)),
        ],
        out_specs=[
            pl.BlockSpec((_BLK, 1), lambda i: (i, 0)),
            pl.BlockSpec((_BLK // _CHUNK, _CHUNK), lambda i: (i, 0)),
            pl.BlockSpec((_K, 128), lambda i: (0, 0)),
            pl.BlockSpec(memory_space=pltpu.SMEM),
        ],
        out_shape=[
            jax.ShapeDtypeStruct((_N, 1), jnp.int32),
            jax.ShapeDtypeStruct((_N // _CHUNK, _CHUNK), jnp.int32),
            jax.ShapeDtypeStruct((_K, 128), jnp.float32),
            jax.ShapeDtypeStruct((1,), jnp.float32),
        ],
    )(x, vectors)


@functools.partial(
    pl.kernel,
    out_type=jax.ShapeDtypeStruct((_N, _D), jnp.float32),
    mesh=plsc.VectorSubcoreMesh(core_axis_name="c", subcore_axis_name="s"),
    scratch_types=[
        pltpu.VMEM((_NCHUNK, _CHUNK), jnp.int32),
        pltpu.VMEM((2, _CHUNK, 128), jnp.float32),
        pltpu.VMEM((_CHUNK, _D), jnp.float32),
        pltpu.SemaphoreType.DMA,
    ],
)
def _sc_gather(table_hbm, idx_hbm, out_hbm, idx_v, rows_v, outc_v, sem):
    wid = lax.axis_index("s") * _NC + lax.axis_index("c")
    base = wid * _BPW
    pltpu.sync_copy(idx_hbm.at[pl.ds(wid * _NCHUNK, _NCHUNK)], idx_v)
    pltpu.async_copy(table_hbm.at[idx_v.at[0]], rows_v.at[0], sem)
    for j in range(_NCHUNK):
        if j + 1 < _NCHUNK:
            pltpu.async_copy(table_hbm.at[idx_v.at[j + 1]],
                             rows_v.at[(j + 1) % 2], sem)
        pltpu.make_async_copy(table_hbm.at[idx_v.at[j]],
                              rows_v.at[j % 2], sem).wait()
        b = j % 2
        for p in range(_CHUNK):
            outc_v[p, pl.ds(0, 16)] = rows_v[b, p, pl.ds(0, 16)]
            outc_v[p, pl.ds(16, 16)] = rows_v[b, p, pl.ds(16, 16)]
        pltpu.sync_copy(outc_v,
                        out_hbm.at[pl.ds(base + j * _CHUNK, _CHUNK)])


def kernel(x, vectors):
    idx, idx2, vt, loss_sum = _tc_part(x, vectors)
    q = _sc_gather(vt, idx2)
    loss = loss_sum[0] / (_N * _D)
    return (q, loss, loss, idx)
